# Initial kernel scaffold; baseline (speedup 1.0000x reference)
import jax
import jax.numpy as jnp
from jax.experimental import pallas as pl


def kernel(user, positive, negative, sub_graph_1_indices, sub_graph_1_values, sub_graph_2_indices, sub_graph_2_values, graph_indices, graph_values, user_emb_weight, item_emb_weight):
    raise NotImplementedError("write your pallas kernel here")


# speedup vs baseline: 1.0002x; 1.0002x over previous
import jax
import jax.numpy as jnp
from jax.experimental import pallas as pl

NUM_USERS = 25000
NUM_ITEMS = 25000
N = NUM_USERS + NUM_ITEMS
D = 64
LAYERS = 3
REG_LAMBDA = 1e-4
SSL_LAMBDA = 0.1
TEMPERATURE = 0.2


def _spmm(indices, values, x):
    row = indices[0]
    col = indices[1]
    gathered = jnp.take(x, col, axis=0) * values[:, None]
    return jax.ops.segment_sum(gathered, row, num_segments=N)


def _agg(indices, values, user_w, item_w):
    all_emb = jnp.concatenate([user_w, item_w], axis=0)
    embeddings = [all_emb]
    for _ in range(LAYERS):
        all_emb = _spmm(indices, values, all_emb)
        embeddings.append(all_emb)
    final = jnp.mean(jnp.stack(embeddings, axis=1), axis=1)
    return final[:NUM_USERS], final[NUM_USERS:]


def _bpr(u, p, n):
    pos_score = jnp.sum(u * p, axis=-1)
    neg_score = jnp.sum(u * n, axis=-1)
    return jnp.mean(jax.nn.softplus(neg_score - pos_score))


def _reg(u, p, n):
    return 0.5 * (jnp.sum(u ** 2) + jnp.sum(p ** 2) + jnp.sum(n ** 2)) / u.shape[0]


def _infonce(v1, v2, temperature):
    v1 = v1 / (jnp.linalg.norm(v1, axis=1, keepdims=True) + 1e-12)
    v2 = v2 / (jnp.linalg.norm(v2, axis=1, keepdims=True) + 1e-12)
    pos_score = jnp.sum(v1 * v2, axis=-1) / temperature
    ttl_score = jax.nn.logsumexp(jnp.matmul(v1, v2.T) / temperature, axis=1)
    return jnp.mean(ttl_score - pos_score)


def kernel(user, positive, negative, sub_graph_1_indices, sub_graph_1_values,
           sub_graph_2_indices, sub_graph_2_values, graph_indices, graph_values,
           user_emb_weight, item_emb_weight):
    all_user_emb, all_item_emb = _agg(graph_indices, graph_values, user_emb_weight, item_emb_weight)
    user_emb_1, item_emb_1 = _agg(sub_graph_1_indices, sub_graph_1_values, user_emb_weight, item_emb_weight)
    user_emb_2, item_emb_2 = _agg(sub_graph_2_indices, sub_graph_2_values, user_emb_weight, item_emb_weight)

    u = jnp.take(all_user_emb, user, axis=0)
    p = jnp.take(all_item_emb, positive, axis=0)
    n = jnp.take(all_item_emb, negative, axis=0)

    ego_u = jnp.take(user_emb_weight, user, axis=0)
    ego_p = jnp.take(item_emb_weight, positive, axis=0)
    ego_n = jnp.take(item_emb_weight, negative, axis=0)

    bpr_loss = _bpr(u, p, n)
    reg_loss = REG_LAMBDA * _reg(ego_u, ego_p, ego_n)

    user_ssl = _infonce(jnp.take(user_emb_1, user, axis=0), jnp.take(user_emb_2, user, axis=0), TEMPERATURE)
    item_ssl = _infonce(jnp.take(item_emb_1, positive, axis=0), jnp.take(item_emb_2, positive, axis=0), TEMPERATURE)
    ssl_loss = SSL_LAMBDA * (user_ssl + item_ssl)

    return jnp.stack([bpr_loss, reg_loss, ssl_loss])


# speedup vs baseline: 7.3931x; 7.3917x over previous
import functools

import jax
import jax.numpy as jnp
from jax import lax
from jax.experimental import pallas as pl
from jax.experimental.pallas import tpu as pltpu
from jax.experimental.pallas import tpu_sc as plsc

NU = 25000
NI = 25000
N = NU + NI
D = 64
DH = 32
LAYERS = 3
REG_LAMBDA = 1e-4
SSL_LAMBDA = 0.1
TEMP = 0.2

NTILES = 16
NP = 50048
RPT = NP // NTILES
W = 512
WR = W // 128


def _spmm3_body(nwin, xcat, colb, rowb, valb, t1, t2, t3,
                cb, rb, vb, gbuf, acc, semI, semG, semS):
    c = lax.axis_index("c")
    s = lax.axis_index("s")
    cN = jnp.full((16,), c * NP, dtype=jnp.int32)

    zero16 = jnp.zeros((16,), jnp.float32)

    def zfill(i, carry):
        gbuf[i, pl.ds(0, 16)] = zero16
        gbuf[i, pl.ds(16, 16)] = zero16
        return carry

    srcs = (xcat, t1, t2)
    dsts = (t1, t2, t3)
    for layer in range(LAYERS):
        src = srcs[layer]
        dst = dsts[layer]

        lax.fori_loop(0, W, zfill, 0)
        for i in range(RPT // W):
            pltpu.sync_copy(gbuf, acc.at[pl.ds(s * RPT + i * W, W)])
        rem = RPT % W
        if rem:
            pltpu.sync_copy(gbuf.at[pl.ds(0, rem)],
                            acc.at[pl.ds(s * RPT + (RPT // W) * W, rem)])
        plsc.subcore_barrier()

        def window(w, carry):
            d1 = pltpu.async_copy(colb.at[s, pl.ds(w * WR, WR)], cb, semI)
            d2 = pltpu.async_copy(rowb.at[s, pl.ds(w * WR, WR)], rb, semI)
            d3 = pltpu.async_copy(valb.at[s, pl.ds(w * WR, WR)], vb, semI)
            d1.wait()
            d2.wait()
            d3.wait()

            def offs(r, cy):
                for k in range(8):
                    cb[r, pl.ds(16 * k, 16)] = cb[r, pl.ds(16 * k, 16)] + cN
                return cy

            lax.fori_loop(0, WR, offs, 0)

            gds = [pltpu.async_copy(src.at[cb.at[j]],
                                    gbuf.at[pl.ds(j * 128, 128)], semG)
                   for j in range(WR)]
            for g in gds:
                g.wait()

            def scale(g, cy):
                j = lax.shift_right_logical(g, 3)
                l16 = lax.bitwise_and(g, 7) * 16
                vvec = vb[j, pl.ds(l16, 16)]
                for k in range(16):
                    e = g * 16 + k
                    vv = lax.gather(
                        vvec, jnp.full((16, 1), k, jnp.int32),
                        lax.GatherDimensionNumbers(offset_dims=(),
                                                   collapsed_slice_dims=(0,),
                                                   start_index_map=(0,)),
                        slice_sizes=(1,),
                        mode=lax.GatherScatterMode.PROMISE_IN_BOUNDS)
                    g0 = gbuf[e, pl.ds(0, 16)]
                    g1 = gbuf[e, pl.ds(16, 16)]
                    gbuf[e, pl.ds(0, 16)] = g0 * vv
                    gbuf[e, pl.ds(16, 16)] = g1 * vv
                return cy

            lax.fori_loop(0, W // 16, scale, 0)

            sds = [pltpu.async_copy(gbuf.at[pl.ds(j * 128, 128)],
                                    acc.at[rb.at[j]], semS, add=True)
                   for j in range(WR)]
            for sd in sds:
                sd.wait()
            return carry

        lax.fori_loop(0, nwin, window, 0)
        plsc.subcore_barrier()

        pltpu.sync_copy(acc.at[pl.ds(s * RPT, RPT)],
                        dst.at[pl.ds(c * NP + s * RPT, RPT)])
        plsc.subcore_barrier()


@functools.cache
def _make_spmm3(nwin):
    mesh = plsc.VectorSubcoreMesh(core_axis_name="c", subcore_axis_name="s")
    out_t = tuple(jax.ShapeDtypeStruct((2 * NP, DH), jnp.float32)
                  for _ in range(LAYERS))
    scratch = [
        pltpu.VMEM((WR, 128), jnp.int32),
        pltpu.VMEM((WR, 128), jnp.int32),
        pltpu.VMEM((WR, 128), jnp.float32),
        pltpu.VMEM((W, DH), jnp.float32),
        pltpu.VMEM_SHARED((NP, DH), jnp.float32),
        pltpu.SemaphoreType.DMA,
        pltpu.SemaphoreType.DMA,
        pltpu.SemaphoreType.DMA,
    ]
    return pl.kernel(functools.partial(_spmm3_body, nwin),
                     out_type=out_t, mesh=mesh, scratch_types=scratch,
                     compiler_params=pltpu.CompilerParams(
                         use_tc_tiling_on_sc=False))


def _aggregate_sc(indices, values, x0, nwin):
    etot = NTILES * nwin * W
    e = values.shape[0]
    pad = etot - e
    fill = jnp.arange(pad, dtype=jnp.int32) % N
    col = jnp.concatenate([indices[1].astype(jnp.int32), fill])
    row = jnp.concatenate([indices[0].astype(jnp.int32), fill])
    val = jnp.concatenate([values, jnp.zeros((pad,), jnp.float32)])
    colb = col.reshape(NTILES, nwin * WR, 128)
    rowb = row.reshape(NTILES, nwin * WR, 128)
    valb = val.reshape(NTILES, nwin * WR, 128)
    xp = jnp.pad(x0, ((0, NP - N), (0, 0)))
    xcat = jnp.concatenate([xp[:, :DH], xp[:, DH:]], axis=0)
    t1, t2, t3 = _make_spmm3(nwin)(xcat, colb, rowb, valb)
    lo = x0[:, :DH] + t1[:N] + t2[:N] + t3[:N]
    hi = x0[:, DH:] + t1[NP:NP + N] + t2[NP:NP + N] + t3[NP:NP + N]
    final = jnp.concatenate([lo, hi], axis=1) * 0.25
    return final[:NU], final[NU:]


def _bpr(u, p, n):
    pos_score = jnp.sum(u * p, axis=-1)
    neg_score = jnp.sum(u * n, axis=-1)
    return jnp.mean(jax.nn.softplus(neg_score - pos_score))


def _reg(u, p, n):
    return 0.5 * (jnp.sum(u ** 2) + jnp.sum(p ** 2) + jnp.sum(n ** 2)) / u.shape[0]


def _infonce(v1, v2, temperature):
    v1 = v1 / (jnp.linalg.norm(v1, axis=1, keepdims=True) + 1e-12)
    v2 = v2 / (jnp.linalg.norm(v2, axis=1, keepdims=True) + 1e-12)
    pos_score = jnp.sum(v1 * v2, axis=-1) / temperature
    ttl_score = jax.nn.logsumexp(jnp.matmul(v1, v2.T) / temperature, axis=1)
    return jnp.mean(ttl_score - pos_score)


def kernel(user, positive, negative, sub_graph_1_indices, sub_graph_1_values,
           sub_graph_2_indices, sub_graph_2_values, graph_indices, graph_values,
           user_emb_weight, item_emb_weight):
    x0 = jnp.concatenate([user_emb_weight, item_emb_weight], axis=0)

    nwin_g = -(-(graph_values.shape[0] // NTILES) // W)
    nwin_s = -(-(sub_graph_1_values.shape[0] // NTILES) // W)

    all_user_emb, all_item_emb = _aggregate_sc(graph_indices, graph_values, x0, nwin_g)
    user_emb_1, item_emb_1 = _aggregate_sc(sub_graph_1_indices, sub_graph_1_values, x0, nwin_s)
    user_emb_2, item_emb_2 = _aggregate_sc(sub_graph_2_indices, sub_graph_2_values, x0, nwin_s)

    u = jnp.take(all_user_emb, user, axis=0)
    p = jnp.take(all_item_emb, positive, axis=0)
    n = jnp.take(all_item_emb, negative, axis=0)

    ego_u = jnp.take(user_emb_weight, user, axis=0)
    ego_p = jnp.take(item_emb_weight, positive, axis=0)
    ego_n = jnp.take(item_emb_weight, negative, axis=0)

    bpr_loss = _bpr(u, p, n)
    reg_loss = REG_LAMBDA * _reg(ego_u, ego_p, ego_n)

    user_ssl = _infonce(jnp.take(user_emb_1, user, axis=0),
                        jnp.take(user_emb_2, user, axis=0), TEMP)
    item_ssl = _infonce(jnp.take(item_emb_1, positive, axis=0),
                        jnp.take(item_emb_2, positive, axis=0), TEMP)
    ssl_loss = SSL_LAMBDA * (user_ssl + item_ssl)

    return jnp.stack([bpr_loss, reg_loss, ssl_loss])


# speedup vs baseline: 8.9679x; 1.2130x over previous
import functools

import jax
import jax.numpy as jnp
from jax import lax
from jax.experimental import pallas as pl
from jax.experimental.pallas import tpu as pltpu
from jax.experimental.pallas import tpu_sc as plsc

NU = 25000
NI = 25000
N = NU + NI
D = 64
DH = 32
LAYERS = 3
REG_LAMBDA = 1e-4
SSL_LAMBDA = 0.1
TEMP = 0.2

NTILES = 16
NP = 50048
RPT = NP // NTILES
W = 384
WR = W // 128

IDXB = 3 * WR * 128 * 4
GATB = W * DH * 4


def _spmm3_body(nwin, xcat, colb, rowb, valb, t1, t2, t3,
                cb0, cb1, cb2, rb0, rb1, rb2, vb0, vb1, vb2,
                g0, g1, acc, semI, semG, semS):
    c = lax.axis_index("c")
    s = lax.axis_index("s")
    cbs = (cb0, cb1, cb2)
    rbs = (rb0, rb1, rb2)
    vbs = (vb0, vb1, vb2)
    gbufs = (g0, g1)

    zero16 = jnp.zeros((16,), jnp.float32)

    def zfill(i, carry):
        g0[i, pl.ds(0, 16)] = zero16
        g0[i, pl.ds(16, 16)] = zero16
        return carry

    def zfill1(i, carry):
        g1[i, pl.ds(0, 16)] = zero16
        g1[i, pl.ds(16, 16)] = zero16
        return carry

    def wait_idx():
        pltpu.make_async_copy(colb.at[c, s, pl.ds(0, WR)], cb0, semI).wait()
        pltpu.make_async_copy(rowb.at[s, pl.ds(0, WR)], rb0, semI).wait()
        pltpu.make_async_copy(valb.at[s, pl.ds(0, WR)], vb0, semI).wait()

    def wait_gathers(src):
        for j in range(WR):
            pltpu.make_async_copy(src.at[pl.ds(0, 128)],
                                  g0.at[pl.ds(j * 128, 128)], semG).wait()

    def wait_scatters(src):
        for j in range(WR):
            pltpu.make_async_copy(src.at[pl.ds(0, 128)],
                                  acc.at[pl.ds(0, 128)], semS).wait()

    def stage_idx(w, q):
        a = pltpu.async_copy(colb.at[c, s, pl.ds(w * WR, WR)], cbs[q], semI)
        b = pltpu.async_copy(rowb.at[s, pl.ds(w * WR, WR)], rbs[q], semI)
        d = pltpu.async_copy(valb.at[s, pl.ds(w * WR, WR)], vbs[q], semI)
        return a, b, d

    def fire_gathers(src, q, p):
        for j in range(WR):
            pltpu.async_copy(src.at[cbs[q].at[j]],
                             gbufs[p].at[pl.ds(j * 128, 128)], semG)

    def fire_scatters(q, p):
        for j in range(WR):
            pltpu.async_copy(gbufs[p].at[pl.ds(j * 128, 128)],
                             acc.at[rbs[q].at[j]], semS, add=True)

    def scale(q, p):
        gb = gbufs[p]
        vb = vbs[q]

        def body(g, cy):
            j = lax.shift_right_logical(g, 3)
            l16 = lax.bitwise_and(g, 7) * 16
            vvec = vb[j, pl.ds(l16, 16)]
            for k in range(16):
                e = g * 16 + k
                vv = lax.gather(
                    vvec, jnp.full((16, 1), k, jnp.int32),
                    lax.GatherDimensionNumbers(offset_dims=(),
                                               collapsed_slice_dims=(0,),
                                               start_index_map=(0,)),
                    slice_sizes=(1,),
                    mode=lax.GatherScatterMode.PROMISE_IN_BOUNDS)
                gb[e, pl.ds(0, 16)] = gb[e, pl.ds(0, 16)] * vv
                gb[e, pl.ds(16, 16)] = gb[e, pl.ds(16, 16)] * vv
            return cy

        lax.fori_loop(0, W // 16, body, 0)

    srcs = (xcat, t1, t2)
    dsts = (t1, t2, t3)
    for layer in range(LAYERS):
        src = srcs[layer]
        dst = dsts[layer]

        lax.fori_loop(0, W, zfill, 0)
        for i in range(RPT // W):
            pltpu.sync_copy(g0, acc.at[pl.ds(s * RPT + i * W, W)])
        rem = RPT % W
        if rem:
            pltpu.sync_copy(g0.at[pl.ds(0, rem)],
                            acc.at[pl.ds(s * RPT + (RPT // W) * W, rem)])
        plsc.subcore_barrier()

        pltpu.sync_copy(colb.at[c, s, pl.ds(0, WR)], cbs[0])
        pltpu.sync_copy(rowb.at[s, pl.ds(0, WR)], rbs[0])
        pltpu.sync_copy(valb.at[s, pl.ds(0, WR)], vbs[0])
        stage_idx(1, 1)
        fire_gathers(src, 0, 0)
        lax.fori_loop(0, W, zfill1, 0)
        fire_scatters(0, 1)

        def superiter(it, carry):
            for woff in range(6):
                w = it * 6 + woff
                p = woff % 2
                q = woff % 3
                q1 = (woff + 1) % 3
                q2 = (woff + 2) % 3
                wait_idx()
                wait_gathers(src)
                wait_scatters(src)
                fire_gathers(src, q1, 1 - p)
                stage_idx(w + 2, q2)
                scale(q, p)
                fire_scatters(q, p)
            return carry

        lax.fori_loop(0, nwin // 6, superiter, 0)

        wait_gathers(src)
        wait_scatters(src)
        wait_idx()
        plsc.subcore_barrier()

        pltpu.sync_copy(acc.at[pl.ds(s * RPT, RPT)],
                        dst.at[pl.ds(c * NP + s * RPT, RPT)])
        plsc.subcore_barrier()


@functools.cache
def _make_spmm3(nwin):
    mesh = plsc.VectorSubcoreMesh(core_axis_name="c", subcore_axis_name="s")
    out_t = tuple(jax.ShapeDtypeStruct((2 * NP, DH), jnp.float32)
                  for _ in range(LAYERS))
    idx_i = pltpu.VMEM((WR, 128), jnp.int32)
    idx_f = pltpu.VMEM((WR, 128), jnp.float32)
    scratch = [
        idx_i, idx_i, idx_i,
        idx_i, idx_i, idx_i,
        idx_f, idx_f, idx_f,
        pltpu.VMEM((W, DH), jnp.float32),
        pltpu.VMEM((W, DH), jnp.float32),
        pltpu.VMEM_SHARED((NP, DH), jnp.float32),
        pltpu.SemaphoreType.DMA,
        pltpu.SemaphoreType.DMA,
        pltpu.SemaphoreType.DMA,
    ]
    return pl.kernel(functools.partial(_spmm3_body, nwin),
                     out_type=out_t, mesh=mesh, scratch_types=scratch,
                     compiler_params=pltpu.CompilerParams(
                         use_tc_tiling_on_sc=False))


def _aggregate_sc(indices, values, x0, nwin):
    etot = NTILES * nwin * W
    e = values.shape[0]
    pad = etot - e
    fill = jnp.arange(pad, dtype=jnp.int32) % N
    col = jnp.concatenate([indices[1].astype(jnp.int32), fill])
    row = jnp.concatenate([indices[0].astype(jnp.int32), fill])
    val = jnp.concatenate([values, jnp.zeros((pad,), jnp.float32)])
    kp = nwin * WR
    colb = col.reshape(NTILES, kp, 128)
    colb = jnp.pad(colb, ((0, 0), (0, 2 * WR), (0, 0)))
    colb = jnp.stack([colb, colb + NP])
    rowb = jnp.pad(row.reshape(NTILES, kp, 128), ((0, 0), (0, 2 * WR), (0, 0)))
    valb = jnp.pad(val.reshape(NTILES, kp, 128), ((0, 0), (0, 2 * WR), (0, 0)))
    xp = jnp.pad(x0, ((0, NP - N), (0, 0)))
    xcat = jnp.concatenate([xp[:, :DH], xp[:, DH:]], axis=0)
    t1, t2, t3 = _make_spmm3(nwin)(xcat, colb, rowb, valb)
    lo = x0[:, :DH] + t1[:N] + t2[:N] + t3[:N]
    hi = x0[:, DH:] + t1[NP:NP + N] + t2[NP:NP + N] + t3[NP:NP + N]
    final = jnp.concatenate([lo, hi], axis=1) * 0.25
    return final[:NU], final[NU:]


def _bpr(u, p, n):
    pos_score = jnp.sum(u * p, axis=-1)
    neg_score = jnp.sum(u * n, axis=-1)
    return jnp.mean(jax.nn.softplus(neg_score - pos_score))


def _reg(u, p, n):
    return 0.5 * (jnp.sum(u ** 2) + jnp.sum(p ** 2) + jnp.sum(n ** 2)) / u.shape[0]


def _infonce(v1, v2, temperature):
    v1 = v1 / (jnp.linalg.norm(v1, axis=1, keepdims=True) + 1e-12)
    v2 = v2 / (jnp.linalg.norm(v2, axis=1, keepdims=True) + 1e-12)
    pos_score = jnp.sum(v1 * v2, axis=-1) / temperature
    ttl_score = jax.nn.logsumexp(jnp.matmul(v1, v2.T) / temperature, axis=1)
    return jnp.mean(ttl_score - pos_score)


def kernel(user, positive, negative, sub_graph_1_indices, sub_graph_1_values,
           sub_graph_2_indices, sub_graph_2_values, graph_indices, graph_values,
           user_emb_weight, item_emb_weight):
    x0 = jnp.concatenate([user_emb_weight, item_emb_weight], axis=0)

    def nwin_for(nedges):
        per_tile = -(-nedges // NTILES)
        return 6 * (-(-per_tile // (6 * W)))

    nwin_g = nwin_for(graph_values.shape[0])
    nwin_s = nwin_for(sub_graph_1_values.shape[0])

    all_user_emb, all_item_emb = _aggregate_sc(graph_indices, graph_values, x0, nwin_g)
    user_emb_1, item_emb_1 = _aggregate_sc(sub_graph_1_indices, sub_graph_1_values, x0, nwin_s)
    user_emb_2, item_emb_2 = _aggregate_sc(sub_graph_2_indices, sub_graph_2_values, x0, nwin_s)

    u = jnp.take(all_user_emb, user, axis=0)
    p = jnp.take(all_item_emb, positive, axis=0)
    n = jnp.take(all_item_emb, negative, axis=0)

    ego_u = jnp.take(user_emb_weight, user, axis=0)
    ego_p = jnp.take(item_emb_weight, positive, axis=0)
    ego_n = jnp.take(item_emb_weight, negative, axis=0)

    bpr_loss = _bpr(u, p, n)
    reg_loss = REG_LAMBDA * _reg(ego_u, ego_p, ego_n)

    user_ssl = _infonce(jnp.take(user_emb_1, user, axis=0),
                        jnp.take(user_emb_2, user, axis=0), TEMP)
    item_ssl = _infonce(jnp.take(item_emb_1, positive, axis=0),
                        jnp.take(item_emb_2, positive, axis=0), TEMP)
    ssl_loss = SSL_LAMBDA * (user_ssl + item_ssl)

    return jnp.stack([bpr_loss, reg_loss, ssl_loss])
